# baseline (speedup 1.0000x reference)
import functools

import jax
import jax.numpy as jnp
from jax import lax
from jax.experimental import pallas as pl
from jax.experimental.pallas import tpu as pltpu
from jax.experimental.pallas import tpu_sc as plsc

N = 10000
NP = 10240
EP = 327680
CH = 128
EPW = EP // 32
NCH = EPW // CH
NC, NS, L = 2, 16, 16
RPT = NP // NS


def _mesh():
    return plsc.VectorSubcoreMesh(
        core_axis_name="c", subcore_axis_name="s",
        num_cores=NC, num_subcores=NS)


_SC_PARAMS = pltpu.CompilerParams(
    needs_layout_passes=False, use_tc_tiling_on_sc=False)


def _rsqrt16(x):
    i = lax.bitcast_convert_type(x, jnp.int32)
    i = jnp.full((L,), 0x5F3759DF, jnp.int32) - lax.shift_right_logical(i, 1)
    y = lax.bitcast_convert_type(i, jnp.float32)
    for _ in range(3):
        y = y * (1.5 - 0.5 * x * y * y)
    return y


def _prep_call(rowf, colf, ewf):
    EPT = EP // NS

    @functools.partial(
        pl.kernel,
        out_type=(jax.ShapeDtypeStruct((EP,), jnp.float32),
                  jax.ShapeDtypeStruct((NP,), jnp.float32)),
        mesh=_mesh(),
        compiler_params=_SC_PARAMS,
        scratch_types=[
            pltpu.VMEM((EPT,), jnp.int32),
            pltpu.VMEM((EPT,), jnp.float32),
            pltpu.VMEM((EPW,), jnp.int32),
            pltpu.VMEM((EPW,), jnp.int32),
            pltpu.VMEM((EPW,), jnp.float32),
            pltpu.VMEM((NP,), jnp.float32),
            pltpu.VMEM((NP,), jnp.float32),
            pltpu.VMEM((RPT,), jnp.float32),
            pltpu.VMEM_SHARED((NP,), jnp.float32),
            pltpu.SemaphoreType.DMA,
        ],
    )
    def prep_kernel(row_hbm, col_hbm, ew_hbm, s_hbm, deg_hbm,
                    dcol_v, dew_v, row_v, col_v, ew_v, deg_v, dinv_v, zb_v,
                    acc_sh, sem):
        cid = lax.axis_index("c")
        sid = lax.axis_index("s")
        wid = cid * NS + sid

        pltpu.sync_copy(col_hbm.at[pl.ds(sid * EPT, EPT)], dcol_v)
        pltpu.sync_copy(ew_hbm.at[pl.ds(sid * EPT, EPT)], dew_v)

        def zb(k, carry):
            zb_v[pl.ds(k * L, L)] = jnp.zeros((L,), jnp.float32)
            return carry
        lax.fori_loop(0, RPT // L, zb, 0)
        pltpu.sync_copy(zb_v, acc_sh.at[pl.ds(sid * RPT, RPT)])
        plsc.subcore_barrier()

        K = 8

        def dchunk(k, carry):
            for j in range(K):
                o = (k * K + j) * CH
                pltpu.async_copy(dew_v.at[pl.ds(o, CH)],
                                 acc_sh.at[dcol_v.at[pl.ds(o, CH)]], sem,
                                 add=True)
            for j in range(K):
                o = (k * K + j) * CH
                pltpu.make_async_copy(
                    dew_v.at[pl.ds(o, CH)],
                    acc_sh.at[dcol_v.at[pl.ds(o, CH)]], sem).wait()
            return carry
        lax.fori_loop(0, EPT // CH // K, dchunk, 0)
        plsc.subcore_barrier()

        @pl.when(cid == 0)
        def _():
            pltpu.sync_copy(acc_sh.at[pl.ds(sid * RPT, RPT)],
                            deg_hbm.at[pl.ds(sid * RPT, RPT)])

        pltpu.sync_copy(acc_sh, deg_v)
        pltpu.sync_copy(row_hbm.at[pl.ds(wid * EPW, EPW)], row_v)
        pltpu.sync_copy(col_hbm.at[pl.ds(wid * EPW, EPW)], col_v)
        pltpu.sync_copy(ew_hbm.at[pl.ds(wid * EPW, EPW)], ew_v)

        def dbody(k, carry):
            sl = pl.ds(k * L, L)
            d = deg_v[sl] + 1.0
            dinv_v[sl] = _rsqrt16(d)
            return carry
        lax.fori_loop(0, NP // L, dbody, 0)

        def nchunk(k, carry):
            for sub in range(4):
                sl = pl.ds(k * 4 * L + sub * L, L)
                rr = row_v[sl]
                cc = col_v[sl]
                w = ew_v[sl]
                ew_v[sl] = (plsc.load_gather(dinv_v, [rr]) * w *
                            plsc.load_gather(dinv_v, [cc]))
            return carry
        lax.fori_loop(0, EPW // (4 * L), nchunk, 0)
        pltpu.sync_copy(ew_v, s_hbm.at[pl.ds(wid * EPW, EPW)])

    return prep_kernel(rowf, colf, ewf)


def _agg_call(F, tables, rowf, colf, sf):
    NT = len(tables)

    @functools.partial(
        pl.kernel,
        out_type=jax.ShapeDtypeStruct((NT, NC, NP, F), jnp.float32),
        mesh=_mesh(),
        compiler_params=_SC_PARAMS,
        scratch_types=[
            pltpu.VMEM((EPW,), jnp.int32),
            pltpu.VMEM((EPW,), jnp.int32),
            pltpu.VMEM((EPW,), jnp.float32),
            pltpu.VMEM((2, CH, F), jnp.float32),
            pltpu.VMEM((2, CH, F), jnp.float32),
            pltpu.VMEM((CH, F), jnp.float32),
            pltpu.VMEM_SHARED((NP, F), jnp.float32),
            pltpu.SemaphoreType.DMA,
            pltpu.SemaphoreType.DMA,
            pltpu.SemaphoreType.DMA,
            pltpu.SemaphoreType.DMA,
        ],
    )
    def agg_kernel(*refs):
        g_hbms = refs[:NT]
        row_hbm, col_hbm, s_hbm, out_hbm = refs[NT:NT + 4]
        (row_v, col_v, s_v, gbuf, sbuf, zbuf, acc_sh,
         sg0, sg1, ss0, ss1) = refs[NT + 4:]
        cid = lax.axis_index("c")
        sid = lax.axis_index("s")
        wid = cid * NS + sid
        semg = (sg0, sg1)
        sems = (ss0, ss1)

        pltpu.sync_copy(row_hbm.at[pl.ds(wid * EPW, EPW)], row_v)
        pltpu.sync_copy(col_hbm.at[pl.ds(wid * EPW, EPW)], col_v)
        pltpu.sync_copy(s_hbm.at[pl.ds(wid * EPW, EPW)], s_v)

        def issue_gather(g_hbm, b, ch):
            pltpu.async_copy(g_hbm.at[row_v.at[pl.ds(ch * CH, CH)]],
                             gbuf.at[b], semg[b])

        def wait_gather(g_hbm, b, ch):
            pltpu.make_async_copy(
                g_hbm.at[row_v.at[pl.ds(ch * CH, CH)]],
                gbuf.at[b], semg[b]).wait()

        def issue_scatter(b, ch):
            pltpu.async_copy(sbuf.at[b],
                             acc_sh.at[col_v.at[pl.ds(ch * CH, CH)]],
                             sems[b], add=True)

        def wait_scatter(b, ch):
            pltpu.make_async_copy(
                sbuf.at[b], acc_sh.at[col_v.at[pl.ds(ch * CH, CH)]],
                sems[b]).wait()

        def scale(b, ch):
            R = 8

            def rbody(r, carry):
                rows = [r * R + rr for rr in range(R)]
                sbs = [plsc.load_gather(
                    s_v, [jnp.full((L,), ch * CH + row, jnp.int32)])
                       for row in rows]
                vals = [[gbuf[b, row, pl.ds(gg * L, L)]
                         for gg in range(F // L)] for row in rows]
                for rr, row in enumerate(rows):
                    for gg in range(F // L):
                        sbuf[b, row, pl.ds(gg * L, L)] = vals[rr][gg] * sbs[rr]
                return carry
            lax.fori_loop(0, CH // R, rbody, 0)

        def zrow(r, carry):
            for gg in range(F // L):
                zbuf[r, pl.ds(gg * L, L)] = jnp.zeros((L,), jnp.float32)
            return carry
        lax.fori_loop(0, CH, zrow, 0)

        for t, g_hbm in enumerate(g_hbms):
            for k in range(RPT // CH):
                pltpu.sync_copy(zbuf,
                                acc_sh.at[pl.ds(sid * RPT + k * CH, CH)])
            plsc.subcore_barrier()

            for b in range(2):
                issue_gather(g_hbm, b, b)
            for b in range(2):
                wait_gather(g_hbm, b, b)
                scale(b, b)
                issue_scatter(b, b)
                issue_gather(g_hbm, b, b + 2)

            def step(k, carry):
                for b in range(2):
                    ch = 2 * k + b
                    wait_gather(g_hbm, b, ch)
                    wait_scatter(b, ch - 2)
                    scale(b, ch)
                    issue_scatter(b, ch)
                    issue_gather(g_hbm, b, ch + 2)
                return carry
            lax.fori_loop(1, NCH // 2 - 1, step, 0)

            for b in range(2):
                ch = NCH - 2 + b
                wait_gather(g_hbm, b, ch)
                wait_scatter(b, ch - 2)
                scale(b, ch)
                issue_scatter(b, ch)
            for b in range(2):
                wait_scatter(b, NCH - 2 + b)

            plsc.subcore_barrier()
            pltpu.sync_copy(acc_sh.at[pl.ds(sid * RPT, RPT)],
                            out_hbm.at[t, cid, pl.ds(sid * RPT, RPT)])
            if t + 1 < NT:
                plsc.subcore_barrier()

    return agg_kernel(*tables, rowf, colf, sf)


def _mm_call(x, w):
    def body(x_ref, w_ref, oa_ref, ob_ref):
        o = jnp.dot(x_ref[...], w_ref[...],
                    preferred_element_type=jnp.float32)
        oa_ref[...] = o[:, :64]
        ob_ref[...] = o[:, 64:]
    return pl.pallas_call(
        body,
        out_shape=(jax.ShapeDtypeStruct((x.shape[0], 64), jnp.float32),
                   jax.ShapeDtypeStruct((x.shape[0], 64), jnp.float32)),
    )(x, w)


def _mid_call(degc, agg1, g1a, g1b, b1r, W2p):
    def body(d_ref, a_ref, ga_ref, gb_ref, b_ref, w_ref, o_ref):
        inv = 1.0 / (d_ref[...] + 1.0)
        agg = jnp.concatenate(
            [a_ref[0, 0] + a_ref[0, 1], a_ref[1, 0] + a_ref[1, 1]], axis=1)
        g = jnp.concatenate([ga_ref[...], gb_ref[...]], axis=1)
        z = agg + g * inv + b_ref[...]
        h = jnp.maximum(z, 0.0)
        o_ref[...] = jnp.dot(h, w_ref[...],
                             preferred_element_type=jnp.float32)
    return pl.pallas_call(
        body,
        out_shape=jax.ShapeDtypeStruct((NP, W2p.shape[1]), jnp.float32),
    )(degc, agg1, g1a, g1b, b1r, W2p)


def _final_call(degc, agg2, g2, b2r):
    F2 = b2r.shape[1]
    def body(d_ref, a_ref, g_ref, b_ref, o_ref):
        inv = 1.0 / (d_ref[...] + 1.0)
        z = ((a_ref[0, 0] + a_ref[0, 1] + g_ref[...] * inv)[:N, :F2]
             + b_ref[...])
        m = jnp.max(z, axis=1, keepdims=True)
        e = jnp.exp(z - m)
        s = jnp.sum(e, axis=1, keepdims=True)
        o_ref[...] = z - m - jnp.log(s)
    return pl.pallas_call(
        body,
        out_shape=jax.ShapeDtypeStruct((N, F2), jnp.float32),
    )(degc, agg2, g2, b2r)


def kernel(x, edge_index, edge_weight, W1, b1, W2, b2):
    row = edge_index[0].astype(jnp.int32)
    col = edge_index[1].astype(jnp.int32)
    ew = edge_weight.astype(jnp.float32)
    pad = EP - row.shape[0]
    pidx = (jnp.arange(pad, dtype=jnp.int32) * 37) % N
    rowf = jnp.concatenate([row, pidx])
    colf = jnp.concatenate([col, pidx])
    ewf = jnp.concatenate([ew, jnp.zeros((pad,), jnp.float32)])
    xp = jnp.concatenate(
        [x, jnp.zeros((NP - N, x.shape[1]), jnp.float32)], axis=0)
    F2P = 48
    W2p = jnp.concatenate(
        [W2, jnp.zeros((W2.shape[0], F2P - W2.shape[1]), jnp.float32)], axis=1)

    sf, deg = _prep_call(rowf, colf, ewf)
    degc = deg.reshape(NP, 1)
    g1a, g1b = _mm_call(xp, W1)
    agg1 = _agg_call(64, [g1a, g1b], rowf, colf, sf)
    g2 = _mid_call(degc, agg1, g1a, g1b,
                   b1.reshape(1, -1), W2p)
    agg2 = _agg_call(F2P, [g2], rowf, colf, sf)
    return _final_call(degc, agg2, g2, b2.reshape(1, -1))


# speedup vs baseline: 1.0152x; 1.0152x over previous
import functools

import jax
import jax.numpy as jnp
from jax import lax
from jax.experimental import pallas as pl
from jax.experimental.pallas import tpu as pltpu
from jax.experimental.pallas import tpu_sc as plsc

N = 10000
NP = 10240
EP = 327680
CH = 128
EPW = EP // 32
NCH = EPW // CH
NC, NS, L = 2, 16, 16
RPT = NP // NS


def _mesh():
    return plsc.VectorSubcoreMesh(
        core_axis_name="c", subcore_axis_name="s",
        num_cores=NC, num_subcores=NS)


_SC_PARAMS = pltpu.CompilerParams(
    needs_layout_passes=False, use_tc_tiling_on_sc=False)


def _rsqrt16(x):
    i = lax.bitcast_convert_type(x, jnp.int32)
    i = jnp.full((L,), 0x5F3759DF, jnp.int32) - lax.shift_right_logical(i, 1)
    y = lax.bitcast_convert_type(i, jnp.float32)
    for _ in range(3):
        y = y * (1.5 - 0.5 * x * y * y)
    return y


def _prep_call(rowf, colf, ewf):
    EPT = EP // NS

    @functools.partial(
        pl.kernel,
        out_type=(jax.ShapeDtypeStruct((EP,), jnp.float32),
                  jax.ShapeDtypeStruct((NP,), jnp.float32)),
        mesh=_mesh(),
        compiler_params=_SC_PARAMS,
        scratch_types=[
            pltpu.VMEM((EPT,), jnp.int32),
            pltpu.VMEM((EPT,), jnp.float32),
            pltpu.VMEM((EPW,), jnp.int32),
            pltpu.VMEM((EPW,), jnp.int32),
            pltpu.VMEM((EPW,), jnp.float32),
            pltpu.VMEM((NP,), jnp.float32),
            pltpu.VMEM((NP,), jnp.float32),
            pltpu.VMEM((RPT,), jnp.float32),
            pltpu.VMEM_SHARED((NP,), jnp.float32),
            pltpu.SemaphoreType.DMA,
        ],
    )
    def prep_kernel(row_hbm, col_hbm, ew_hbm, s_hbm, deg_hbm,
                    dcol_v, dew_v, row_v, col_v, ew_v, deg_v, dinv_v, zb_v,
                    acc_sh, sem):
        cid = lax.axis_index("c")
        sid = lax.axis_index("s")
        wid = cid * NS + sid

        pltpu.sync_copy(col_hbm.at[pl.ds(sid * EPT, EPT)], dcol_v)
        pltpu.sync_copy(ew_hbm.at[pl.ds(sid * EPT, EPT)], dew_v)

        def zb(k, carry):
            zb_v[pl.ds(k * L, L)] = jnp.zeros((L,), jnp.float32)
            return carry
        lax.fori_loop(0, RPT // L, zb, 0)
        pltpu.sync_copy(zb_v, acc_sh.at[pl.ds(sid * RPT, RPT)])
        plsc.subcore_barrier()

        K = 8

        def dchunk(k, carry):
            for j in range(K):
                o = (k * K + j) * CH
                pltpu.async_copy(dew_v.at[pl.ds(o, CH)],
                                 acc_sh.at[dcol_v.at[pl.ds(o, CH)]], sem,
                                 add=True)
            for j in range(K):
                o = (k * K + j) * CH
                pltpu.make_async_copy(
                    dew_v.at[pl.ds(o, CH)],
                    acc_sh.at[dcol_v.at[pl.ds(o, CH)]], sem).wait()
            return carry
        lax.fori_loop(0, EPT // CH // K, dchunk, 0)
        plsc.subcore_barrier()

        @pl.when(cid == 0)
        def _():
            pltpu.sync_copy(acc_sh.at[pl.ds(sid * RPT, RPT)],
                            deg_hbm.at[pl.ds(sid * RPT, RPT)])

        pltpu.sync_copy(acc_sh, deg_v)
        pltpu.sync_copy(row_hbm.at[pl.ds(wid * EPW, EPW)], row_v)
        pltpu.sync_copy(col_hbm.at[pl.ds(wid * EPW, EPW)], col_v)
        pltpu.sync_copy(ew_hbm.at[pl.ds(wid * EPW, EPW)], ew_v)

        def dbody(k, carry):
            sl = pl.ds(k * L, L)
            d = deg_v[sl] + 1.0
            dinv_v[sl] = _rsqrt16(d)
            return carry
        lax.fori_loop(0, NP // L, dbody, 0)

        def nchunk(k, carry):
            for sub in range(4):
                sl = pl.ds(k * 4 * L + sub * L, L)
                rr = row_v[sl]
                cc = col_v[sl]
                w = ew_v[sl]
                ew_v[sl] = (plsc.load_gather(dinv_v, [rr]) * w *
                            plsc.load_gather(dinv_v, [cc]))
            return carry
        lax.fori_loop(0, EPW // (4 * L), nchunk, 0)
        pltpu.sync_copy(ew_v, s_hbm.at[pl.ds(wid * EPW, EPW)])

    return prep_kernel(rowf, colf, ewf)


def _agg_call(F, tables, rowf, colf, sf):
    NT = len(tables)

    @functools.partial(
        pl.kernel,
        out_type=jax.ShapeDtypeStruct((NT, NC, NP, F), jnp.float32),
        mesh=_mesh(),
        compiler_params=_SC_PARAMS,
        scratch_types=[
            pltpu.VMEM((EPW,), jnp.int32),
            pltpu.VMEM((EPW,), jnp.int32),
            pltpu.VMEM((EPW,), jnp.float32),
            pltpu.VMEM((2, CH, F), jnp.float32),
            pltpu.VMEM((2, CH, F), jnp.float32),
            pltpu.VMEM((CH, F), jnp.float32),
            pltpu.VMEM_SHARED((NP, F), jnp.float32),
            pltpu.SemaphoreType.DMA,
            pltpu.SemaphoreType.DMA,
            pltpu.SemaphoreType.DMA,
            pltpu.SemaphoreType.DMA,
        ],
    )
    def agg_kernel(*refs):
        g_hbms = refs[:NT]
        row_hbm, col_hbm, s_hbm, out_hbm = refs[NT:NT + 4]
        (row_v, col_v, s_v, gbuf, sbuf, zbuf, acc_sh,
         sg0, sg1, ss0, ss1) = refs[NT + 4:]
        cid = lax.axis_index("c")
        sid = lax.axis_index("s")
        wid = cid * NS + sid
        semg = (sg0, sg1)
        sems = (ss0, ss1)

        pltpu.sync_copy(row_hbm.at[pl.ds(wid * EPW, EPW)], row_v)
        pltpu.sync_copy(col_hbm.at[pl.ds(wid * EPW, EPW)], col_v)
        pltpu.sync_copy(s_hbm.at[pl.ds(wid * EPW, EPW)], s_v)

        def issue_gather(g_hbm, b, ch):
            pltpu.async_copy(g_hbm.at[row_v.at[pl.ds(ch * CH, CH)]],
                             gbuf.at[b], semg[b])

        def wait_gather(g_hbm, b, ch):
            pltpu.make_async_copy(
                g_hbm.at[row_v.at[pl.ds(ch * CH, CH)]],
                gbuf.at[b], semg[b]).wait()

        def issue_scatter(b, ch):
            pltpu.async_copy(sbuf.at[b],
                             acc_sh.at[col_v.at[pl.ds(ch * CH, CH)]],
                             sems[b], add=True)

        def wait_scatter(b, ch):
            pltpu.make_async_copy(
                sbuf.at[b], acc_sh.at[col_v.at[pl.ds(ch * CH, CH)]],
                sems[b]).wait()

        def scale(b, ch):
            R = 4

            def rbody(r, carry):
                rows = [r * R + rr for rr in range(R)]
                sbs = [plsc.load_gather(
                    s_v, [jnp.full((L,), ch * CH + row, jnp.int32)])
                       for row in rows]
                vals = [[gbuf[b, row, pl.ds(gg * L, L)]
                         for gg in range(F // L)] for row in rows]
                for rr, row in enumerate(rows):
                    for gg in range(F // L):
                        sbuf[b, row, pl.ds(gg * L, L)] = vals[rr][gg] * sbs[rr]
                return carry
            lax.fori_loop(0, CH // R, rbody, 0)

        def zrow(r, carry):
            for gg in range(F // L):
                zbuf[r, pl.ds(gg * L, L)] = jnp.zeros((L,), jnp.float32)
            return carry
        lax.fori_loop(0, CH, zrow, 0)

        for t, g_hbm in enumerate(g_hbms):
            for k in range(RPT // CH):
                pltpu.sync_copy(zbuf,
                                acc_sh.at[pl.ds(sid * RPT + k * CH, CH)])
            plsc.subcore_barrier()

            for b in range(2):
                issue_gather(g_hbm, b, b)
            for b in range(2):
                wait_gather(g_hbm, b, b)
                scale(b, b)
                issue_scatter(b, b)
                issue_gather(g_hbm, b, b + 2)

            def step(k, carry):
                for b in range(2):
                    ch = 2 * k + b
                    wait_gather(g_hbm, b, ch)
                    wait_scatter(b, ch - 2)
                    scale(b, ch)
                    issue_scatter(b, ch)
                    issue_gather(g_hbm, b, ch + 2)
                return carry
            lax.fori_loop(1, NCH // 2 - 1, step, 0)

            for b in range(2):
                ch = NCH - 2 + b
                wait_gather(g_hbm, b, ch)
                wait_scatter(b, ch - 2)
                scale(b, ch)
                issue_scatter(b, ch)
            for b in range(2):
                wait_scatter(b, NCH - 2 + b)

            plsc.subcore_barrier()
            pltpu.sync_copy(acc_sh.at[pl.ds(sid * RPT, RPT)],
                            out_hbm.at[t, cid, pl.ds(sid * RPT, RPT)])
            if t + 1 < NT:
                plsc.subcore_barrier()

    return agg_kernel(*tables, rowf, colf, sf)


def _mm_call(x, w):
    def body(x_ref, w_ref, oa_ref, ob_ref):
        o = jnp.dot(x_ref[...], w_ref[...],
                    preferred_element_type=jnp.float32)
        oa_ref[...] = o[:, :64]
        ob_ref[...] = o[:, 64:]
    return pl.pallas_call(
        body,
        out_shape=(jax.ShapeDtypeStruct((x.shape[0], 64), jnp.float32),
                   jax.ShapeDtypeStruct((x.shape[0], 64), jnp.float32)),
    )(x, w)


def _mid_call(degc, agg1, g1a, g1b, b1r, W2p):
    def body(d_ref, a_ref, ga_ref, gb_ref, b_ref, w_ref, o_ref):
        inv = 1.0 / (d_ref[...] + 1.0)
        agg = jnp.concatenate(
            [a_ref[0, 0] + a_ref[0, 1], a_ref[1, 0] + a_ref[1, 1]], axis=1)
        g = jnp.concatenate([ga_ref[...], gb_ref[...]], axis=1)
        z = agg + g * inv + b_ref[...]
        h = jnp.maximum(z, 0.0)
        o_ref[...] = jnp.dot(h, w_ref[...],
                             preferred_element_type=jnp.float32)
    return pl.pallas_call(
        body,
        out_shape=jax.ShapeDtypeStruct((NP, W2p.shape[1]), jnp.float32),
    )(degc, agg1, g1a, g1b, b1r, W2p)


def _final_call(degc, agg2, g2, b2r):
    F2 = b2r.shape[1]
    def body(d_ref, a_ref, g_ref, b_ref, o_ref):
        inv = 1.0 / (d_ref[...] + 1.0)
        z = ((a_ref[0, 0] + a_ref[0, 1] + g_ref[...] * inv)[:N, :F2]
             + b_ref[...])
        m = jnp.max(z, axis=1, keepdims=True)
        e = jnp.exp(z - m)
        s = jnp.sum(e, axis=1, keepdims=True)
        o_ref[...] = z - m - jnp.log(s)
    return pl.pallas_call(
        body,
        out_shape=jax.ShapeDtypeStruct((N, F2), jnp.float32),
    )(degc, agg2, g2, b2r)


def kernel(x, edge_index, edge_weight, W1, b1, W2, b2):
    row = edge_index[0].astype(jnp.int32)
    col = edge_index[1].astype(jnp.int32)
    ew = edge_weight.astype(jnp.float32)
    pad = EP - row.shape[0]
    pidx = (jnp.arange(pad, dtype=jnp.int32) * 37) % N
    rowf = jnp.concatenate([row, pidx])
    colf = jnp.concatenate([col, pidx])
    ewf = jnp.concatenate([ew, jnp.zeros((pad,), jnp.float32)])
    xp = jnp.concatenate(
        [x, jnp.zeros((NP - N, x.shape[1]), jnp.float32)], axis=0)
    F2P = 48
    W2p = jnp.concatenate(
        [W2, jnp.zeros((W2.shape[0], F2P - W2.shape[1]), jnp.float32)], axis=1)

    sf, deg = _prep_call(rowf, colf, ewf)
    degc = deg.reshape(NP, 1)
    g1a, g1b = _mm_call(xp, W1)
    agg1 = _agg_call(64, [g1a, g1b], rowf, colf, sf)
    g2 = _mid_call(degc, agg1, g1a, g1b,
                   b1.reshape(1, -1), W2p)
    agg2 = _agg_call(F2P, [g2], rowf, colf, sf)
    return _final_call(degc, agg2, g2, b2.reshape(1, -1))


# speedup vs baseline: 1.1350x; 1.1180x over previous
import functools

import jax
import jax.numpy as jnp
from jax import lax
from jax.experimental import pallas as pl
from jax.experimental.pallas import tpu as pltpu
from jax.experimental.pallas import tpu_sc as plsc

N = 10000
NP = 10240
EP = 327680
CH = 128
EPW = EP // 32
NCH = EPW // CH
NC, NS, L = 2, 16, 16
RPT = NP // NS


def _mesh():
    return plsc.VectorSubcoreMesh(
        core_axis_name="c", subcore_axis_name="s",
        num_cores=NC, num_subcores=NS)


_SC_PARAMS = pltpu.CompilerParams(
    needs_layout_passes=False, use_tc_tiling_on_sc=False)


def _rsqrt16(x):
    i = lax.bitcast_convert_type(x, jnp.int32)
    i = jnp.full((L,), 0x5F3759DF, jnp.int32) - lax.shift_right_logical(i, 1)
    y = lax.bitcast_convert_type(i, jnp.float32)
    for _ in range(3):
        y = y * (1.5 - 0.5 * x * y * y)
    return y


def _prep_call(rowf, colf, ewf):
    EPT = EP // NS

    @functools.partial(
        pl.kernel,
        out_type=(jax.ShapeDtypeStruct((EP,), jnp.float32),
                  jax.ShapeDtypeStruct((NP,), jnp.float32)),
        mesh=_mesh(),
        compiler_params=_SC_PARAMS,
        scratch_types=[
            pltpu.VMEM((EPT,), jnp.int32),
            pltpu.VMEM((EPT,), jnp.float32),
            pltpu.VMEM((EPW,), jnp.int32),
            pltpu.VMEM((EPW,), jnp.int32),
            pltpu.VMEM((EPW,), jnp.float32),
            pltpu.VMEM((NP,), jnp.float32),
            pltpu.VMEM((NP,), jnp.float32),
            pltpu.VMEM((RPT,), jnp.float32),
            pltpu.VMEM_SHARED((NP,), jnp.float32),
            pltpu.SemaphoreType.DMA,
        ],
    )
    def prep_kernel(row_hbm, col_hbm, ew_hbm, s_hbm, deg_hbm,
                    dcol_v, dew_v, row_v, col_v, ew_v, deg_v, dinv_v, zb_v,
                    acc_sh, sem):
        cid = lax.axis_index("c")
        sid = lax.axis_index("s")
        wid = cid * NS + sid

        pltpu.sync_copy(col_hbm.at[pl.ds(sid * EPT, EPT)], dcol_v)
        pltpu.sync_copy(ew_hbm.at[pl.ds(sid * EPT, EPT)], dew_v)

        def zb(k, carry):
            zb_v[pl.ds(k * L, L)] = jnp.zeros((L,), jnp.float32)
            return carry
        lax.fori_loop(0, RPT // L, zb, 0)
        pltpu.sync_copy(zb_v, acc_sh.at[pl.ds(sid * RPT, RPT)])
        plsc.subcore_barrier()

        K = 8

        def dchunk(k, carry):
            for j in range(K):
                o = (k * K + j) * CH
                pltpu.async_copy(dew_v.at[pl.ds(o, CH)],
                                 acc_sh.at[dcol_v.at[pl.ds(o, CH)]], sem,
                                 add=True)
            for j in range(K):
                o = (k * K + j) * CH
                pltpu.make_async_copy(
                    dew_v.at[pl.ds(o, CH)],
                    acc_sh.at[dcol_v.at[pl.ds(o, CH)]], sem).wait()
            return carry
        lax.fori_loop(0, EPT // CH // K, dchunk, 0)
        plsc.subcore_barrier()

        @pl.when(cid == 0)
        def _():
            pltpu.sync_copy(acc_sh.at[pl.ds(sid * RPT, RPT)],
                            deg_hbm.at[pl.ds(sid * RPT, RPT)])

        pltpu.sync_copy(acc_sh, deg_v)
        pltpu.sync_copy(row_hbm.at[pl.ds(wid * EPW, EPW)], row_v)
        pltpu.sync_copy(col_hbm.at[pl.ds(wid * EPW, EPW)], col_v)
        pltpu.sync_copy(ew_hbm.at[pl.ds(wid * EPW, EPW)], ew_v)

        def dbody(k, carry):
            sl = pl.ds(k * L, L)
            d = deg_v[sl] + 1.0
            dinv_v[sl] = _rsqrt16(d)
            return carry
        lax.fori_loop(0, NP // L, dbody, 0)

        def nchunk(k, carry):
            for sub in range(4):
                sl = pl.ds(k * 4 * L + sub * L, L)
                rr = row_v[sl]
                cc = col_v[sl]
                w = ew_v[sl]
                ew_v[sl] = (plsc.load_gather(dinv_v, [rr]) * w *
                            plsc.load_gather(dinv_v, [cc]))
            return carry
        lax.fori_loop(0, EPW // (4 * L), nchunk, 0)
        pltpu.sync_copy(ew_v, s_hbm.at[pl.ds(wid * EPW, EPW)])

    return prep_kernel(rowf, colf, ewf)


def _agg_call(F, tables, rowf, colf, sf):
    NT = len(tables)

    @functools.partial(
        pl.kernel,
        out_type=jax.ShapeDtypeStruct((NT, NC, NP, F), jnp.float32),
        mesh=_mesh(),
        compiler_params=_SC_PARAMS,
        scratch_types=[
            pltpu.VMEM((EPW,), jnp.int32),
            pltpu.VMEM((EPW,), jnp.int32),
            pltpu.VMEM((EPW,), jnp.float32),
            pltpu.VMEM((3, CH, F), jnp.float32),
            pltpu.VMEM((2, CH, F), jnp.float32),
            pltpu.VMEM((CH, F), jnp.float32),
            pltpu.VMEM_SHARED((NP, F), jnp.float32),
            pltpu.SemaphoreType.DMA,
            pltpu.SemaphoreType.DMA,
            pltpu.SemaphoreType.DMA,
            pltpu.SemaphoreType.DMA,
            pltpu.SemaphoreType.DMA,
        ],
    )
    def agg_kernel(*refs):
        g_hbms = refs[:NT]
        row_hbm, col_hbm, s_hbm, out_hbm = refs[NT:NT + 4]
        (row_v, col_v, s_v, gbuf, sbuf, zbuf, acc_sh,
         sg0, sg1, sg2, ss0, ss1) = refs[NT + 4:]
        cid = lax.axis_index("c")
        sid = lax.axis_index("s")
        wid = cid * NS + sid
        semg = (sg0, sg1, sg2)
        sems = (ss0, ss1)

        pltpu.sync_copy(row_hbm.at[pl.ds(wid * EPW, EPW)], row_v)
        pltpu.sync_copy(col_hbm.at[pl.ds(wid * EPW, EPW)], col_v)
        pltpu.sync_copy(s_hbm.at[pl.ds(wid * EPW, EPW)], s_v)

        def issue_gather(g_hbm, b, ch):
            pltpu.async_copy(g_hbm.at[row_v.at[pl.ds(ch * CH, CH)]],
                             gbuf.at[b], semg[b])

        def wait_gather(g_hbm, b, ch):
            pltpu.make_async_copy(
                g_hbm.at[row_v.at[pl.ds(ch * CH, CH)]],
                gbuf.at[b], semg[b]).wait()

        def issue_scatter(b, ch):
            pltpu.async_copy(sbuf.at[b],
                             acc_sh.at[col_v.at[pl.ds(ch * CH, CH)]],
                             sems[b], add=True)

        def wait_scatter(b, ch):
            pltpu.make_async_copy(
                sbuf.at[b], acc_sh.at[col_v.at[pl.ds(ch * CH, CH)]],
                sems[b]).wait()

        def scale3(gb, sb, ch):
            R = 4

            def rbody(r, carry):
                rows = [r * R + rr for rr in range(R)]
                sbs = [plsc.load_gather(
                    s_v, [jnp.full((L,), ch * CH + row, jnp.int32)])
                       for row in rows]
                vals = [[gbuf[gb, row, pl.ds(gg * L, L)]
                         for gg in range(F // L)] for row in rows]
                for rr, row in enumerate(rows):
                    for gg in range(F // L):
                        sbuf[sb, row, pl.ds(gg * L, L)] = vals[rr][gg] * sbs[rr]
                return carry
            lax.fori_loop(0, CH // R, rbody, 0)

        def zrow(r, carry):
            for gg in range(F // L):
                zbuf[r, pl.ds(gg * L, L)] = jnp.zeros((L,), jnp.float32)
            return carry
        lax.fori_loop(0, CH, zrow, 0)

        for t, g_hbm in enumerate(g_hbms):
            for k in range(RPT // CH):
                pltpu.sync_copy(zbuf,
                                acc_sh.at[pl.ds(sid * RPT + k * CH, CH)])
            plsc.subcore_barrier()

            for ch in range(2):
                issue_gather(g_hbm, ch % 3, ch)
            for ch in range(6):
                issue_gather(g_hbm, (ch + 2) % 3, ch + 2)
                wait_gather(g_hbm, ch % 3, ch)
                if ch >= 2:
                    wait_scatter(ch % 2, ch - 2)
                scale3(ch % 3, ch % 2, ch)
                issue_scatter(ch % 2, ch)

            def step(k, carry):
                for j in range(6):
                    ch = 6 * k + j
                    issue_gather(g_hbm, (j + 2) % 3, ch + 2)
                    wait_gather(g_hbm, j % 3, ch)
                    wait_scatter(j % 2, ch - 2)
                    scale3(j % 3, j % 2, ch)
                    issue_scatter(j % 2, ch)
                return carry
            lax.fori_loop(1, (NCH - 2) // 6, step, 0)

            for ch in range(NCH - 2, NCH):
                wait_gather(g_hbm, ch % 3, ch)
                wait_scatter(ch % 2, ch - 2)
                scale3(ch % 3, ch % 2, ch)
                issue_scatter(ch % 2, ch)
            for ch in range(NCH - 2, NCH):
                wait_scatter(ch % 2, ch)

            plsc.subcore_barrier()
            pltpu.sync_copy(acc_sh.at[pl.ds(sid * RPT, RPT)],
                            out_hbm.at[t, cid, pl.ds(sid * RPT, RPT)])
            if t + 1 < NT:
                plsc.subcore_barrier()

    return agg_kernel(*tables, rowf, colf, sf)


def _mm_call(x, w):
    def body(x_ref, w_ref, oa_ref, ob_ref):
        o = jnp.dot(x_ref[...], w_ref[...],
                    preferred_element_type=jnp.float32)
        oa_ref[...] = o[:, :64]
        ob_ref[...] = o[:, 64:]
    return pl.pallas_call(
        body,
        out_shape=(jax.ShapeDtypeStruct((x.shape[0], 64), jnp.float32),
                   jax.ShapeDtypeStruct((x.shape[0], 64), jnp.float32)),
    )(x, w)


def _mid_call(degc, agg1, g1a, g1b, b1r, W2p):
    def body(d_ref, a_ref, ga_ref, gb_ref, b_ref, w_ref, o_ref):
        inv = 1.0 / (d_ref[...] + 1.0)
        agg = jnp.concatenate(
            [a_ref[0, 0] + a_ref[0, 1], a_ref[1, 0] + a_ref[1, 1]], axis=1)
        g = jnp.concatenate([ga_ref[...], gb_ref[...]], axis=1)
        z = agg + g * inv + b_ref[...]
        h = jnp.maximum(z, 0.0)
        o_ref[...] = jnp.dot(h, w_ref[...],
                             preferred_element_type=jnp.float32)
    return pl.pallas_call(
        body,
        out_shape=jax.ShapeDtypeStruct((NP, W2p.shape[1]), jnp.float32),
    )(degc, agg1, g1a, g1b, b1r, W2p)


def _final_call(degc, agg2, g2, b2r):
    F2 = b2r.shape[1]
    def body(d_ref, a_ref, g_ref, b_ref, o_ref):
        inv = 1.0 / (d_ref[...] + 1.0)
        z = ((a_ref[0, 0] + a_ref[0, 1] + g_ref[...] * inv)[:N, :F2]
             + b_ref[...])
        m = jnp.max(z, axis=1, keepdims=True)
        e = jnp.exp(z - m)
        s = jnp.sum(e, axis=1, keepdims=True)
        o_ref[...] = z - m - jnp.log(s)
    return pl.pallas_call(
        body,
        out_shape=jax.ShapeDtypeStruct((N, F2), jnp.float32),
    )(degc, agg2, g2, b2r)


def kernel(x, edge_index, edge_weight, W1, b1, W2, b2):
    row = edge_index[0].astype(jnp.int32)
    col = edge_index[1].astype(jnp.int32)
    ew = edge_weight.astype(jnp.float32)
    pad = EP - row.shape[0]
    pidx = (jnp.arange(pad, dtype=jnp.int32) * 37) % N
    rowf = jnp.concatenate([row, pidx])
    colf = jnp.concatenate([col, pidx])
    ewf = jnp.concatenate([ew, jnp.zeros((pad,), jnp.float32)])
    xp = jnp.concatenate(
        [x, jnp.zeros((NP - N, x.shape[1]), jnp.float32)], axis=0)
    F2P = 48
    W2p = jnp.concatenate(
        [W2, jnp.zeros((W2.shape[0], F2P - W2.shape[1]), jnp.float32)], axis=1)

    sf, deg = _prep_call(rowf, colf, ewf)
    degc = deg.reshape(NP, 1)
    g1a, g1b = _mm_call(xp, W1)
    agg1 = _agg_call(64, [g1a, g1b], rowf, colf, sf)
    g2 = _mid_call(degc, agg1, g1a, g1b,
                   b1.reshape(1, -1), W2p)
    agg2 = _agg_call(F2P, [g2], rowf, colf, sf)
    return _final_call(degc, agg2, g2, b2.reshape(1, -1))


# speedup vs baseline: 1.1524x; 1.0153x over previous
import functools

import jax
import jax.numpy as jnp
from jax import lax
from jax.experimental import pallas as pl
from jax.experimental.pallas import tpu as pltpu
from jax.experimental.pallas import tpu_sc as plsc

N = 10000
NP = 10240
EP = 327680
CH = 128
EPW = EP // 32
NCH = EPW // CH
NC, NS, L = 2, 16, 16
RPT = NP // NS


def _mesh():
    return plsc.VectorSubcoreMesh(
        core_axis_name="c", subcore_axis_name="s",
        num_cores=NC, num_subcores=NS)


_SC_PARAMS = pltpu.CompilerParams(
    needs_layout_passes=False, use_tc_tiling_on_sc=False)


def _rsqrt16(x):
    i = lax.bitcast_convert_type(x, jnp.int32)
    i = jnp.full((L,), 0x5F3759DF, jnp.int32) - lax.shift_right_logical(i, 1)
    y = lax.bitcast_convert_type(i, jnp.float32)
    for _ in range(3):
        y = y * (1.5 - 0.5 * x * y * y)
    return y


def _prep_call(rowf, colf, ewf):
    EPT = EP // NS

    @functools.partial(
        pl.kernel,
        out_type=(jax.ShapeDtypeStruct((EP,), jnp.float32),
                  jax.ShapeDtypeStruct((NP,), jnp.float32)),
        mesh=_mesh(),
        compiler_params=_SC_PARAMS,
        scratch_types=[
            pltpu.VMEM((EPT,), jnp.int32),
            pltpu.VMEM((EPT,), jnp.float32),
            pltpu.VMEM((EPW,), jnp.int32),
            pltpu.VMEM((EPW,), jnp.int32),
            pltpu.VMEM((EPW,), jnp.float32),
            pltpu.VMEM((NP,), jnp.float32),
            pltpu.VMEM((NP,), jnp.float32),
            pltpu.VMEM((RPT,), jnp.float32),
            pltpu.VMEM_SHARED((NP,), jnp.float32),
            pltpu.SemaphoreType.DMA,
        ],
    )
    def prep_kernel(row_hbm, col_hbm, ew_hbm, s_hbm, deg_hbm,
                    dcol_v, dew_v, row_v, col_v, ew_v, deg_v, dinv_v, zb_v,
                    acc_sh, sem):
        cid = lax.axis_index("c")
        sid = lax.axis_index("s")
        wid = cid * NS + sid

        pltpu.sync_copy(col_hbm.at[pl.ds(sid * EPT, EPT)], dcol_v)
        pltpu.sync_copy(ew_hbm.at[pl.ds(sid * EPT, EPT)], dew_v)

        def zb(k, carry):
            zb_v[pl.ds(k * L, L)] = jnp.zeros((L,), jnp.float32)
            return carry
        lax.fori_loop(0, RPT // L, zb, 0)
        pltpu.sync_copy(zb_v, acc_sh.at[pl.ds(sid * RPT, RPT)])
        plsc.subcore_barrier()

        K = 8

        def dchunk(k, carry):
            for j in range(K):
                o = (k * K + j) * CH
                pltpu.async_copy(dew_v.at[pl.ds(o, CH)],
                                 acc_sh.at[dcol_v.at[pl.ds(o, CH)]], sem,
                                 add=True)
            for j in range(K):
                o = (k * K + j) * CH
                pltpu.make_async_copy(
                    dew_v.at[pl.ds(o, CH)],
                    acc_sh.at[dcol_v.at[pl.ds(o, CH)]], sem).wait()
            return carry
        lax.fori_loop(0, EPT // CH // K, dchunk, 0)
        plsc.subcore_barrier()

        @pl.when(cid == 0)
        def _():
            pltpu.sync_copy(acc_sh.at[pl.ds(sid * RPT, RPT)],
                            deg_hbm.at[pl.ds(sid * RPT, RPT)])

        pltpu.sync_copy(acc_sh, deg_v)
        pltpu.sync_copy(row_hbm.at[pl.ds(wid * EPW, EPW)], row_v)
        pltpu.sync_copy(col_hbm.at[pl.ds(wid * EPW, EPW)], col_v)
        pltpu.sync_copy(ew_hbm.at[pl.ds(wid * EPW, EPW)], ew_v)

        def dbody(k, carry):
            sl = pl.ds(k * L, L)
            d = deg_v[sl] + 1.0
            dinv_v[sl] = _rsqrt16(d)
            return carry
        lax.fori_loop(0, NP // L, dbody, 0)

        def nchunk(k, carry):
            for sub in range(4):
                sl = pl.ds(k * 4 * L + sub * L, L)
                rr = row_v[sl]
                cc = col_v[sl]
                w = ew_v[sl]
                ew_v[sl] = (plsc.load_gather(dinv_v, [rr]) * w *
                            plsc.load_gather(dinv_v, [cc]))
            return carry
        lax.fori_loop(0, EPW // (4 * L), nchunk, 0)
        pltpu.sync_copy(ew_v, s_hbm.at[pl.ds(wid * EPW, EPW)])

    return prep_kernel(rowf, colf, ewf)


def _agg_call(F, tables, rowf, colf, sf):
    NT = len(tables)

    @functools.partial(
        pl.kernel,
        out_type=jax.ShapeDtypeStruct((NT, NC, NP, F), jnp.float32),
        mesh=_mesh(),
        compiler_params=_SC_PARAMS,
        scratch_types=[
            pltpu.VMEM((EPW,), jnp.int32),
            pltpu.VMEM((EPW,), jnp.int32),
            pltpu.VMEM((EPW,), jnp.float32),
            pltpu.VMEM((3, CH, F), jnp.float32),
            pltpu.VMEM((2, CH, F), jnp.float32),
            pltpu.VMEM((CH, F), jnp.float32),
            pltpu.VMEM_SHARED((NP, F), jnp.float32),
            pltpu.SemaphoreType.DMA,
            pltpu.SemaphoreType.DMA,
            pltpu.SemaphoreType.DMA,
            pltpu.SemaphoreType.DMA,
            pltpu.SemaphoreType.DMA,
        ],
    )
    def agg_kernel(*refs):
        g_hbms = refs[:NT]
        row_hbm, col_hbm, s_hbm, out_hbm = refs[NT:NT + 4]
        (row_v, col_v, s_v, gbuf, sbuf, zbuf, acc_sh,
         sg0, sg1, sg2, ss0, ss1) = refs[NT + 4:]
        cid = lax.axis_index("c")
        sid = lax.axis_index("s")
        wid = cid * NS + sid
        semg = (sg0, sg1, sg2)
        sems = (ss0, ss1)

        pltpu.sync_copy(row_hbm.at[pl.ds(wid * EPW, EPW)], row_v)
        pltpu.sync_copy(col_hbm.at[pl.ds(wid * EPW, EPW)], col_v)
        pltpu.sync_copy(s_hbm.at[pl.ds(wid * EPW, EPW)], s_v)

        def issue_gather(g_hbm, b, ch):
            pltpu.async_copy(g_hbm.at[row_v.at[pl.ds(ch * CH, CH)]],
                             gbuf.at[b], semg[b])

        def wait_gather(g_hbm, b, ch):
            pltpu.make_async_copy(
                g_hbm.at[row_v.at[pl.ds(ch * CH, CH)]],
                gbuf.at[b], semg[b]).wait()

        def issue_scatter(b, ch):
            pltpu.async_copy(sbuf.at[b],
                             acc_sh.at[col_v.at[pl.ds(ch * CH, CH)]],
                             sems[b], add=True)

        def wait_scatter(b, ch):
            pltpu.make_async_copy(
                sbuf.at[b], acc_sh.at[col_v.at[pl.ds(ch * CH, CH)]],
                sems[b]).wait()

        def scale3(gb, sb, ch):
            R = 4

            def rbody(r, carry):
                rows = [r * R + rr for rr in range(R)]
                sbs = [plsc.load_gather(
                    s_v, [jnp.full((L,), ch * CH + row, jnp.int32)])
                       for row in rows]
                vals = [[gbuf[gb, row, pl.ds(gg * L, L)]
                         for gg in range(F // L)] for row in rows]
                for rr, row in enumerate(rows):
                    for gg in range(F // L):
                        sbuf[sb, row, pl.ds(gg * L, L)] = vals[rr][gg] * sbs[rr]
                return carry
            lax.fori_loop(0, CH // R, rbody, 0)

        def zrow(r, carry):
            for gg in range(F // L):
                zbuf[r, pl.ds(gg * L, L)] = jnp.zeros((L,), jnp.float32)
            return carry
        lax.fori_loop(0, CH, zrow, 0)

        for t, g_hbm in enumerate(g_hbms):
            for k in range(RPT // CH):
                pltpu.sync_copy(zbuf,
                                acc_sh.at[pl.ds(sid * RPT + k * CH, CH)])
            plsc.subcore_barrier()

            for ch in range(2):
                issue_gather(g_hbm, ch % 3, ch)
            for ch in range(6):
                issue_gather(g_hbm, (ch + 2) % 3, ch + 2)
                wait_gather(g_hbm, ch % 3, ch)
                if ch >= 2:
                    wait_scatter(ch % 2, ch - 2)
                scale3(ch % 3, ch % 2, ch)
                issue_scatter(ch % 2, ch)

            def step(k, carry):
                for j in range(6):
                    ch = 6 * k + j
                    issue_gather(g_hbm, (j + 2) % 3, ch + 2)
                    wait_gather(g_hbm, j % 3, ch)
                    wait_scatter(j % 2, ch - 2)
                    scale3(j % 3, j % 2, ch)
                    issue_scatter(j % 2, ch)
                return carry
            lax.fori_loop(1, (NCH - 2) // 6, step, 0)

            for ch in range(NCH - 2, NCH):
                wait_gather(g_hbm, ch % 3, ch)
                wait_scatter(ch % 2, ch - 2)
                scale3(ch % 3, ch % 2, ch)
                issue_scatter(ch % 2, ch)
            for ch in range(NCH - 2, NCH):
                wait_scatter(ch % 2, ch)

            plsc.subcore_barrier()
            pltpu.sync_copy(acc_sh.at[pl.ds(sid * RPT, RPT)],
                            out_hbm.at[t, cid, pl.ds(sid * RPT, RPT)])
            if t + 1 < NT:
                plsc.subcore_barrier()

    return agg_kernel(*tables, rowf, colf, sf)


def _mm_call(x, w):
    BM = 2048
    def body(x_ref, w_ref, oa_ref, ob_ref):
        o = jnp.dot(x_ref[...], w_ref[...],
                    preferred_element_type=jnp.float32)
        oa_ref[...] = o[:, :64]
        ob_ref[...] = o[:, 64:]
    return pl.pallas_call(
        body,
        grid=(NP // BM,),
        in_specs=[pl.BlockSpec((BM, 128), lambda i: (i, 0)),
                  pl.BlockSpec((128, 128), lambda i: (0, 0))],
        out_specs=(pl.BlockSpec((BM, 64), lambda i: (i, 0)),
                   pl.BlockSpec((BM, 64), lambda i: (i, 0))),
        out_shape=(jax.ShapeDtypeStruct((x.shape[0], 64), jnp.float32),
                   jax.ShapeDtypeStruct((x.shape[0], 64), jnp.float32)),
    )(x, w)


def _mid_call(degc, agg1, g1a, g1b, b1r, W2p):
    BM = 2048
    def body(d_ref, a_ref, ga_ref, gb_ref, b_ref, w_ref, o_ref):
        inv = 1.0 / (d_ref[...] + 1.0)
        agg = jnp.concatenate(
            [a_ref[0, 0] + a_ref[0, 1], a_ref[1, 0] + a_ref[1, 1]], axis=1)
        g = jnp.concatenate([ga_ref[...], gb_ref[...]], axis=1)
        z = agg + g * inv + b_ref[...]
        h = jnp.maximum(z, 0.0)
        o_ref[...] = jnp.dot(h, w_ref[...],
                             preferred_element_type=jnp.float32)
    F2 = W2p.shape[1]
    return pl.pallas_call(
        body,
        grid=(NP // BM,),
        in_specs=[pl.BlockSpec((BM, 1), lambda i: (i, 0)),
                  pl.BlockSpec((2, 2, BM, 64), lambda i: (0, 0, i, 0)),
                  pl.BlockSpec((BM, 64), lambda i: (i, 0)),
                  pl.BlockSpec((BM, 64), lambda i: (i, 0)),
                  pl.BlockSpec((1, 128), lambda i: (0, 0)),
                  pl.BlockSpec((128, F2), lambda i: (0, 0))],
        out_specs=pl.BlockSpec((BM, F2), lambda i: (i, 0)),
        out_shape=jax.ShapeDtypeStruct((NP, F2), jnp.float32),
    )(degc, agg1, g1a, g1b, b1r, W2p)


def _final_call(degc, agg2, g2, b2r):
    F2 = b2r.shape[1]
    BM = 2000
    F2P = g2.shape[1]
    def body(d_ref, a_ref, g_ref, b_ref, o_ref):
        inv = 1.0 / (d_ref[...] + 1.0)
        z = ((a_ref[0, 0] + a_ref[0, 1] + g_ref[...] * inv)[:, :F2]
             + b_ref[...])
        m = jnp.max(z, axis=1, keepdims=True)
        e = jnp.exp(z - m)
        s = jnp.sum(e, axis=1, keepdims=True)
        o_ref[...] = z - m - jnp.log(s)
    return pl.pallas_call(
        body,
        grid=(N // BM,),
        in_specs=[pl.BlockSpec((BM, 1), lambda i: (i, 0)),
                  pl.BlockSpec((1, 2, BM, F2P), lambda i: (0, 0, i, 0)),
                  pl.BlockSpec((BM, F2P), lambda i: (i, 0)),
                  pl.BlockSpec((1, F2), lambda i: (0, 0))],
        out_specs=pl.BlockSpec((BM, F2), lambda i: (i, 0)),
        out_shape=jax.ShapeDtypeStruct((N, F2), jnp.float32),
    )(degc, agg2, g2, b2r)


def kernel(x, edge_index, edge_weight, W1, b1, W2, b2):
    row = edge_index[0].astype(jnp.int32)
    col = edge_index[1].astype(jnp.int32)
    ew = edge_weight.astype(jnp.float32)
    pad = EP - row.shape[0]
    pidx = (jnp.arange(pad, dtype=jnp.int32) * 37) % N
    rowf = jnp.concatenate([row, pidx])
    colf = jnp.concatenate([col, pidx])
    ewf = jnp.concatenate([ew, jnp.zeros((pad,), jnp.float32)])
    xp = jnp.concatenate(
        [x, jnp.zeros((NP - N, x.shape[1]), jnp.float32)], axis=0)
    F2P = 48
    W2p = jnp.concatenate(
        [W2, jnp.zeros((W2.shape[0], F2P - W2.shape[1]), jnp.float32)], axis=1)

    sf, deg = _prep_call(rowf, colf, ewf)
    degc = deg.reshape(NP, 1)
    g1a, g1b = _mm_call(xp, W1)
    agg1 = _agg_call(64, [g1a, g1b], rowf, colf, sf)
    g2 = _mid_call(degc, agg1, g1a, g1b,
                   b1.reshape(1, -1), W2p)
    agg2 = _agg_call(F2P, [g2], rowf, colf, sf)
    return _final_call(degc, agg2, g2, b2.reshape(1, -1))
